# baseline (speedup 1.0000x reference)
import functools

import jax
import jax.numpy as jnp
import numpy as np
from jax import lax
from jax.experimental import pallas as pl
from jax.experimental.pallas import tpu as pltpu
from jax.experimental.pallas import tpu_sc as plsc

_BN = np.float32(1.0 / np.sqrt(1.0 + 1e-5))
_N, _E, _IN, _H, _OUT = 10000, 160000, 64, 256, 2
_HH = _H // 2
_RB = 1000
_EB = 80
_NT = 16
_EPT = _E // _NT
_ZR = _N // _NT
_SIGMA = np.float32(np.deg2rad(10.0) / np.sqrt(3.0))


def _relu_bn(v, b):
    return jnp.maximum((v + b[...]) * _BN, 0.0)


def _mm(a, w):
    return jnp.dot(a, w, preferred_element_type=jnp.float32)


def _k1_body(x_ref, wi0, bi0, wi1, bi1, wa, ba, m_ref, h_ref):
    h = _relu_bn(_mm(x_ref[...], wi0[...]), bi0)
    h = _relu_bn(_mm(h, wi1[...]), bi1)
    m = _relu_bn(_mm(h, wa[...]), ba)
    h_ref[...] = h
    m_ref[0] = m[:, :_HH]
    m_ref[1] = m[:, _HH:]


def _ks_body(h_ref, ws, bs, s_ref):
    s_ref[...] = _relu_bn(_mm(h_ref[...], ws[...]), bs)


def _k2_body(s_ref, agg_ref, wc, bc, wa, ba, m_ref, h_ref):
    acc = _mm(s_ref[...], wc[:_H, :])
    acc += _mm(agg_ref[0], wc[_H:_H + _HH, :])
    acc += _mm(agg_ref[1], wc[_H + _HH:, :])
    h = _relu_bn(acc, bc)
    m = _relu_bn(_mm(h, wa[...]), ba)
    h_ref[...] = h
    m_ref[0] = m[:, :_HH]
    m_ref[1] = m[:, _HH:]


def _k3_body(s_ref, agg_ref, wc, bc, wr, br, out_ref):
    acc = _mm(s_ref[...], wc[:_H, :])
    acc += _mm(agg_ref[0], wc[_H:_H + _HH, :])
    acc += _mm(agg_ref[1], wc[_H + _HH:, :])
    h = _relu_bn(acc, bc)
    f = _mm(h, wr[...]) + br[...]
    out_ref[...] = jax.nn.sigmoid(f) * (6.0 * _SIGMA) - 3.0 * _SIGMA


def _rowspec(shape):
    nd = len(shape)
    return pl.BlockSpec(shape, lambda i: (0,) * nd)


_GRID = (_N // _RB,)
_W_HH = _rowspec((_IN, _H))
_W_HHH = _rowspec((_H, _H))
_W_C = _rowspec((2 * _H, _H))
_B_H = _rowspec((1, _H))
_S_SPEC = pl.BlockSpec((_RB, _H), lambda i: (i, 0))
_M_SPEC = pl.BlockSpec((2, _RB, _HH), lambda i: (0, i, 0))

_k1 = pl.pallas_call(
    _k1_body,
    grid=_GRID,
    in_specs=[pl.BlockSpec((_RB, _IN), lambda i: (i, 0)),
              _W_HH, _B_H, _W_HHH, _B_H, _W_HHH, _B_H],
    out_specs=[_M_SPEC, _S_SPEC],
    out_shape=[jax.ShapeDtypeStruct((2, _N, _HH), jnp.float32),
               jax.ShapeDtypeStruct((_N, _H), jnp.float32)],
)

_ks = pl.pallas_call(
    _ks_body,
    grid=_GRID,
    in_specs=[_S_SPEC, _W_HHH, _B_H],
    out_specs=_S_SPEC,
    out_shape=jax.ShapeDtypeStruct((_N, _H), jnp.float32),
)

_k2 = pl.pallas_call(
    _k2_body,
    grid=_GRID,
    in_specs=[_S_SPEC, _M_SPEC, _W_C, _B_H, _W_HHH, _B_H],
    out_specs=[_M_SPEC, _S_SPEC],
    out_shape=[jax.ShapeDtypeStruct((2, _N, _HH), jnp.float32),
               jax.ShapeDtypeStruct((_N, _H), jnp.float32)],
)

_k3 = pl.pallas_call(
    _k3_body,
    grid=_GRID,
    in_specs=[_S_SPEC, _M_SPEC, _W_C, _B_H,
              _rowspec((_H, _OUT)), _rowspec((1, _OUT))],
    out_specs=pl.BlockSpec((_RB, _OUT), lambda i: (i, 0)),
    out_shape=jax.ShapeDtypeStruct((_N, _OUT), jnp.float32),
)


_NB = 2
_NEC = _EPT // _EB


def _seg_body(m_hbm, src_hbm, dst_hbm, out_hbm, acc, sva, dva,
              rows0, rows1, gsem0, gsem1, ssem0, ssem1):
    c = lax.axis_index("c")
    s = lax.axis_index("s")
    rows_l = [rows0, rows1]
    gsems_l = [gsem0, gsem1]
    ssems_l = [ssem0, ssem1]

    pltpu.sync_copy(src_hbm.at[c, s], sva)
    pltpu.sync_copy(dst_hbm.at[s], dva)

    def _zb(i, carry):
        for j in range(8):
            rows0[i, pl.ds(j * 16, 16)] = jnp.zeros((16,), jnp.float32)
        return carry
    lax.fori_loop(0, _EB, _zb, 0)

    nch = _N // _EB
    nit = (nch + _NT - 1) // _NT

    def _zc(i, carry):
        ch = i * _NT + s

        @pl.when(ch < nch)
        def _():
            pltpu.async_copy(rows0, acc.at[pl.ds(ch * _EB, _EB)], gsem0)
        return carry
    lax.fori_loop(0, nit, _zc, 0)

    def _zd(i, carry):
        ch = i * _NT + s

        @pl.when(ch < nch)
        def _():
            pltpu.make_async_copy(rows0, acc.at[pl.ds(ch * _EB, _EB)],
                                  gsem0).wait()
        return carry
    lax.fori_loop(0, nit, _zd, 0)
    plsc.subcore_barrier()

    def _gat(k, rb, gsem):
        return pltpu.make_async_copy(
            m_hbm.at[sva.at[pl.ds(k * _EB, _EB)]], rb, gsem)

    def _visit(k, j, jo):
        @pl.when((k >= 2) & (k <= _NEC + 1))
        def _():
            pltpu.make_async_copy(rows_l[j], acc.at[dva.at[k - 2]],
                                  ssems_l[j]).wait()

        @pl.when(k < _NEC)
        def _():
            _gat(k, rows_l[j], gsems_l[j]).start()

        @pl.when((k >= 1) & (k <= _NEC))
        def _():
            _gat(k - 1, rows_l[jo], gsems_l[jo]).wait()
            pltpu.async_copy(rows_l[jo], acc.at[dva.at[k - 1]],
                             ssems_l[jo], add=True)

    def _eb(i, carry):
        _visit(2 * i, 0, 1)
        _visit(2 * i + 1, 1, 0)
        return carry
    lax.fori_loop(0, (_NEC + 3) // 2 + 1, _eb, 0)
    plsc.subcore_barrier()

    def _wc(i, carry):
        ch = i * _NT + s

        @pl.when(ch < nch)
        def _():
            pltpu.async_copy(acc.at[pl.ds(ch * _EB, _EB)],
                             out_hbm.at[pl.ds(c * _N + ch * _EB, _EB)],
                             gsem0)
        return carry
    lax.fori_loop(0, nit, _wc, 0)

    def _wd(i, carry):
        ch = i * _NT + s

        @pl.when(ch < nch)
        def _():
            pltpu.make_async_copy(
                acc.at[pl.ds(ch * _EB, _EB)],
                out_hbm.at[pl.ds(c * _N + ch * _EB, _EB)], gsem0).wait()
        return carry
    lax.fori_loop(0, nit, _wd, 0)


@functools.cache
def _make_seg_sum():
    return pl.kernel(
        _seg_body,
        out_type=jax.ShapeDtypeStruct((2 * _N, _HH), jnp.float32),
        mesh=plsc.VectorSubcoreMesh(core_axis_name="c", subcore_axis_name="s",
                                    num_cores=2, num_subcores=_NT),
        scratch_types=[
            pltpu.VMEM_SHARED((_N, _HH), jnp.float32),
            pltpu.VMEM((_EPT,), jnp.int32),
            pltpu.VMEM((_NEC, _EB), jnp.int32),
            pltpu.VMEM((_EB, _HH), jnp.float32),
            pltpu.VMEM((_EB, _HH), jnp.float32),
            pltpu.SemaphoreType.DMA,
            pltpu.SemaphoreType.DMA,
            pltpu.SemaphoreType.DMA,
            pltpu.SemaphoreType.DMA,
        ],
    )


def _seg_sum(m2, src2, dst):
    return _make_seg_sum()(m2, src2, dst)


def kernel(x, edge_index, batch_size, Wi0, bi0, Wi1, bi1, Wa, ba, Ws, bs,
           Wc, bc, Wr, br):
    src = edge_index[0].astype(jnp.int32)
    dst = edge_index[1].astype(jnp.int32)
    src2 = jnp.stack([src, src + jnp.int32(_N)]).reshape(2, _NT, _EPT)
    dst2 = dst.reshape(_NT, _NEC, _EB)

    bi0r = bi0.reshape(1, _H)
    bi1r = bi1.reshape(1, _H)
    bar = ba.reshape(2, 1, _H)
    bsr = bs.reshape(2, 1, _H)
    bcr = bc.reshape(2, 1, _H)
    brr = br.reshape(1, _OUT)

    m, h = _k1(x, Wi0, bi0r, Wi1, bi1r, Wa[0], bar[0])
    agg = _seg_sum(m.reshape(2 * _N, _HH), src2, dst2).reshape(2, _N, _HH)
    s = _ks(h, Ws[0], bsr[0])
    m, h = _k2(s, agg, Wc[0], bcr[0], Wa[1], bar[1])
    agg = _seg_sum(m.reshape(2 * _N, _HH), src2, dst2).reshape(2, _N, _HH)
    s = _ks(h, Ws[1], bsr[1])
    out = _k3(s, agg, Wc[1], bcr[1], Wr, brr)
    return out.reshape(100, _N // 100, _OUT)


# speedup vs baseline: 1.0158x; 1.0158x over previous
import functools

import jax
import jax.numpy as jnp
import numpy as np
from jax import lax
from jax.experimental import pallas as pl
from jax.experimental.pallas import tpu as pltpu
from jax.experimental.pallas import tpu_sc as plsc

_BN = np.float32(1.0 / np.sqrt(1.0 + 1e-5))
_N, _E, _IN, _H, _OUT = 10000, 160000, 64, 256, 2
_HH = _H // 2
_RB = 1000
_EB = 80
_NT = 16
_EPT = _E // _NT
_ZR = _N // _NT
_SIGMA = np.float32(np.deg2rad(10.0) / np.sqrt(3.0))


def _relu_bn(v, b):
    return jnp.maximum((v + b[...]) * _BN, 0.0)


def _mm(a, w):
    return jnp.dot(a, w, preferred_element_type=jnp.float32)


def _k1_body(x_ref, wi0, bi0, wi1, bi1, wa, ba, ws, bs, m_ref, s_ref):
    h = _relu_bn(_mm(x_ref[...], wi0[...]), bi0)
    h = _relu_bn(_mm(h, wi1[...]), bi1)
    m = _relu_bn(_mm(h, wa[...]), ba)
    s_ref[...] = _relu_bn(_mm(h, ws[...]), bs)
    m_ref[0] = m[:, :_HH]
    m_ref[1] = m[:, _HH:]


def _k2_body(s_ref, agg_ref, wc, bc, wa, ba, ws, bs, m_ref, s_out_ref):
    acc = _mm(s_ref[...], wc[:_H, :])
    acc += _mm(agg_ref[0], wc[_H:_H + _HH, :])
    acc += _mm(agg_ref[1], wc[_H + _HH:, :])
    h = _relu_bn(acc, bc)
    m = _relu_bn(_mm(h, wa[...]), ba)
    s_out_ref[...] = _relu_bn(_mm(h, ws[...]), bs)
    m_ref[0] = m[:, :_HH]
    m_ref[1] = m[:, _HH:]


def _k3_body(s_ref, agg_ref, wc, bc, wr, br, out_ref):
    acc = _mm(s_ref[...], wc[:_H, :])
    acc += _mm(agg_ref[0], wc[_H:_H + _HH, :])
    acc += _mm(agg_ref[1], wc[_H + _HH:, :])
    h = _relu_bn(acc, bc)
    f = _mm(h, wr[...]) + br[...]
    out_ref[...] = jax.nn.sigmoid(f) * (6.0 * _SIGMA) - 3.0 * _SIGMA


def _rowspec(shape):
    nd = len(shape)
    return pl.BlockSpec(shape, lambda i: (0,) * nd)


_GRID = (_N // _RB,)
_W_HH = _rowspec((_IN, _H))
_W_HHH = _rowspec((_H, _H))
_W_C = _rowspec((2 * _H, _H))
_B_H = _rowspec((1, _H))
_S_SPEC = pl.BlockSpec((_RB, _H), lambda i: (i, 0))
_M_SPEC = pl.BlockSpec((2, _RB, _HH), lambda i: (0, i, 0))

_k1 = pl.pallas_call(
    _k1_body,
    grid=_GRID,
    in_specs=[pl.BlockSpec((_RB, _IN), lambda i: (i, 0)),
              _W_HH, _B_H, _W_HHH, _B_H, _W_HHH, _B_H, _W_HHH, _B_H],
    out_specs=[_M_SPEC, _S_SPEC],
    out_shape=[jax.ShapeDtypeStruct((2, _N, _HH), jnp.float32),
               jax.ShapeDtypeStruct((_N, _H), jnp.float32)],
)

_k2 = pl.pallas_call(
    _k2_body,
    grid=_GRID,
    in_specs=[_S_SPEC, _M_SPEC, _W_C, _B_H, _W_HHH, _B_H, _W_HHH, _B_H],
    out_specs=[_M_SPEC, _S_SPEC],
    out_shape=[jax.ShapeDtypeStruct((2, _N, _HH), jnp.float32),
               jax.ShapeDtypeStruct((_N, _H), jnp.float32)],
)

_k3 = pl.pallas_call(
    _k3_body,
    grid=_GRID,
    in_specs=[_S_SPEC, _M_SPEC, _W_C, _B_H,
              _rowspec((_H, _OUT)), _rowspec((1, _OUT))],
    out_specs=pl.BlockSpec((_RB, _OUT), lambda i: (i, 0)),
    out_shape=jax.ShapeDtypeStruct((_N, _OUT), jnp.float32),
)


_NB = 2
_NEC = _EPT // _EB


def _seg_body(m_hbm, src_hbm, dst_hbm, out_hbm, acc, sva, dva,
              rows0, rows1, gsem0, gsem1, ssem0, ssem1):
    c = lax.axis_index("c")
    s = lax.axis_index("s")
    rows_l = [rows0, rows1]
    gsems_l = [gsem0, gsem1]
    ssems_l = [ssem0, ssem1]

    pltpu.sync_copy(src_hbm.at[c, s], sva)
    pltpu.sync_copy(dst_hbm.at[s], dva)

    def _zb(i, carry):
        for j in range(8):
            rows0[i, pl.ds(j * 16, 16)] = jnp.zeros((16,), jnp.float32)
        return carry
    lax.fori_loop(0, _EB, _zb, 0)

    nch = _N // _EB
    nit = (nch + _NT - 1) // _NT

    def _zc(i, carry):
        ch = i * _NT + s

        @pl.when(ch < nch)
        def _():
            pltpu.async_copy(rows0, acc.at[pl.ds(ch * _EB, _EB)], gsem0)
        return carry
    lax.fori_loop(0, nit, _zc, 0)

    def _zd(i, carry):
        ch = i * _NT + s

        @pl.when(ch < nch)
        def _():
            pltpu.make_async_copy(rows0, acc.at[pl.ds(ch * _EB, _EB)],
                                  gsem0).wait()
        return carry
    lax.fori_loop(0, nit, _zd, 0)
    plsc.subcore_barrier()

    def _gat(k, rb, gsem):
        return pltpu.make_async_copy(
            m_hbm.at[sva.at[pl.ds(k * _EB, _EB)]], rb, gsem)

    def _visit(k, j, jo):
        @pl.when((k >= 2) & (k <= _NEC + 1))
        def _():
            pltpu.make_async_copy(rows_l[j], acc.at[dva.at[k - 2]],
                                  ssems_l[j]).wait()

        @pl.when(k < _NEC)
        def _():
            _gat(k, rows_l[j], gsems_l[j]).start()

        @pl.when((k >= 1) & (k <= _NEC))
        def _():
            _gat(k - 1, rows_l[jo], gsems_l[jo]).wait()
            pltpu.async_copy(rows_l[jo], acc.at[dva.at[k - 1]],
                             ssems_l[jo], add=True)

    def _eb(i, carry):
        _visit(2 * i, 0, 1)
        _visit(2 * i + 1, 1, 0)
        return carry
    lax.fori_loop(0, (_NEC + 3) // 2 + 1, _eb, 0)
    plsc.subcore_barrier()

    def _wc(i, carry):
        ch = i * _NT + s

        @pl.when(ch < nch)
        def _():
            pltpu.async_copy(acc.at[pl.ds(ch * _EB, _EB)],
                             out_hbm.at[pl.ds(c * _N + ch * _EB, _EB)],
                             gsem0)
        return carry
    lax.fori_loop(0, nit, _wc, 0)

    def _wd(i, carry):
        ch = i * _NT + s

        @pl.when(ch < nch)
        def _():
            pltpu.make_async_copy(
                acc.at[pl.ds(ch * _EB, _EB)],
                out_hbm.at[pl.ds(c * _N + ch * _EB, _EB)], gsem0).wait()
        return carry
    lax.fori_loop(0, nit, _wd, 0)


@functools.cache
def _make_seg_sum():
    return pl.kernel(
        _seg_body,
        out_type=jax.ShapeDtypeStruct((2 * _N, _HH), jnp.float32),
        mesh=plsc.VectorSubcoreMesh(core_axis_name="c", subcore_axis_name="s",
                                    num_cores=2, num_subcores=_NT),
        scratch_types=[
            pltpu.VMEM_SHARED((_N, _HH), jnp.float32),
            pltpu.VMEM((_EPT,), jnp.int32),
            pltpu.VMEM((_NEC, _EB), jnp.int32),
            pltpu.VMEM((_EB, _HH), jnp.float32),
            pltpu.VMEM((_EB, _HH), jnp.float32),
            pltpu.SemaphoreType.DMA,
            pltpu.SemaphoreType.DMA,
            pltpu.SemaphoreType.DMA,
            pltpu.SemaphoreType.DMA,
        ],
    )


def _seg_sum(m2, src2, dst):
    return _make_seg_sum()(m2, src2, dst)


def kernel(x, edge_index, batch_size, Wi0, bi0, Wi1, bi1, Wa, ba, Ws, bs,
           Wc, bc, Wr, br):
    src = edge_index[0].astype(jnp.int32)
    dst = edge_index[1].astype(jnp.int32)
    src2 = jnp.stack([src, src + jnp.int32(_N)]).reshape(2, _NT, _EPT)
    dst2 = dst.reshape(_NT, _NEC, _EB)

    bi0r = bi0.reshape(1, _H)
    bi1r = bi1.reshape(1, _H)
    bar = ba.reshape(2, 1, _H)
    bsr = bs.reshape(2, 1, _H)
    bcr = bc.reshape(2, 1, _H)
    brr = br.reshape(1, _OUT)

    m, s = _k1(x, Wi0, bi0r, Wi1, bi1r, Wa[0], bar[0], Ws[0], bsr[0])
    agg = _seg_sum(m.reshape(2 * _N, _HH), src2, dst2).reshape(2, _N, _HH)
    m, s = _k2(s, agg, Wc[0], bcr[0], Wa[1], bar[1], Ws[1], bsr[1])
    agg = _seg_sum(m.reshape(2 * _N, _HH), src2, dst2).reshape(2, _N, _HH)
    out = _k3(s, agg, Wc[1], bcr[1], Wr, brr)
    return out.reshape(100, _N // 100, _OUT)
